# Initial kernel scaffold; baseline (speedup 1.0000x reference)
import jax
import jax.numpy as jnp
from jax.experimental import pallas as pl


def kernel(feature):
    raise NotImplementedError("write your pallas kernel here")


# speedup vs baseline: 1.3964x; 1.3964x over previous
import jax
import jax.numpy as jnp
from jax.experimental import pallas as pl
from jax.experimental.pallas import tpu as pltpu

_NUM_BINS = 64
_CHUNK = 1024
_COLS = 8


def _minmax_kernel(x_ref, lo_ref, hi_ref):
    x = x_ref[...]
    lo_ref[0, 0] = jnp.min(x)
    hi_ref[0, 0] = jnp.max(x)


def _expand_kernel(lo_ref, hi_ref, xt_ref, out_ref):
    lo = lo_ref[0, 0]
    hi = hi_ref[0, 0]
    delta = (hi - lo) / jnp.float32(_NUM_BINS - 1)
    inv = jnp.float32(1.0) / delta
    xblk = xt_ref[0]
    t = (xblk - lo) * inv
    idx = jnp.clip(jnp.floor(t).astype(jnp.int32) + 1, 0, _NUM_BINS)
    cols = jax.lax.broadcasted_iota(jnp.int32, (_CHUNK, _NUM_BINS + 1), 1)
    for c in range(_COLS):
        icol = jax.lax.slice_in_dim(idx, c, c + 1, axis=1)
        out_ref[c * _CHUNK:(c + 1) * _CHUNK, :] = (
            icol == cols).astype(jnp.float32)


def kernel(feature):
    n = feature.shape[0]
    f2 = feature.reshape(n // 128, 128)
    lo, hi = pl.pallas_call(
        _minmax_kernel,
        out_shape=(
            jax.ShapeDtypeStruct((1, 1), jnp.float32),
            jax.ShapeDtypeStruct((1, 1), jnp.float32),
        ),
        out_specs=(
            pl.BlockSpec(memory_space=pltpu.SMEM),
            pl.BlockSpec(memory_space=pltpu.SMEM),
        ),
    )(f2)

    n_chunks = n // _CHUNK
    grid = n_chunks // _COLS
    rows_per_step = _CHUNK * _COLS
    xt = feature.reshape(grid, _COLS, _CHUNK).transpose(0, 2, 1)
    out = pl.pallas_call(
        _expand_kernel,
        grid=(grid,),
        in_specs=[
            pl.BlockSpec((1, 1), lambda g: (0, 0), memory_space=pltpu.SMEM),
            pl.BlockSpec((1, 1), lambda g: (0, 0), memory_space=pltpu.SMEM),
            pl.BlockSpec((1, _CHUNK, _COLS), lambda g: (g, 0, 0)),
        ],
        out_specs=pl.BlockSpec((rows_per_step, _NUM_BINS + 1),
                               lambda g: (g, 0)),
        out_shape=jax.ShapeDtypeStruct((n, _NUM_BINS + 1), jnp.float32),
    )(lo, hi, xt)
    return out


# speedup vs baseline: 1.5305x; 1.0961x over previous
import jax
import jax.numpy as jnp
from jax.experimental import pallas as pl
from jax.experimental.pallas import tpu as pltpu

_NUM_BINS = 64
_CHUNK = 1024
_COLS = 16


def _minmax_kernel(x_ref, lo_ref, hi_ref):
    x = x_ref[...]
    lo_ref[0, 0] = jnp.min(x)
    hi_ref[0, 0] = jnp.max(x)


def _expand_kernel(lo_ref, hi_ref, xt_ref, out_ref):
    lo = lo_ref[0, 0]
    hi = hi_ref[0, 0]
    delta = (hi - lo) / jnp.float32(_NUM_BINS - 1)
    inv = jnp.float32(1.0) / delta
    xblk = xt_ref[0]
    t = (xblk - lo) * inv
    idx = jnp.clip(jnp.floor(t).astype(jnp.int32) + 1, 0, _NUM_BINS)
    cols = jax.lax.broadcasted_iota(jnp.int32, (_CHUNK, _NUM_BINS + 1), 1)
    for c in range(_COLS):
        icol = jax.lax.slice_in_dim(idx, c, c + 1, axis=1)
        out_ref[c * _CHUNK:(c + 1) * _CHUNK, :] = (
            icol == cols).astype(jnp.float32)


def kernel(feature):
    n = feature.shape[0]
    f2 = feature.reshape(n // 128, 128)
    lo, hi = pl.pallas_call(
        _minmax_kernel,
        out_shape=(
            jax.ShapeDtypeStruct((1, 1), jnp.float32),
            jax.ShapeDtypeStruct((1, 1), jnp.float32),
        ),
        out_specs=(
            pl.BlockSpec(memory_space=pltpu.SMEM),
            pl.BlockSpec(memory_space=pltpu.SMEM),
        ),
    )(f2)

    n_chunks = n // _CHUNK
    grid = n_chunks // _COLS
    rows_per_step = _CHUNK * _COLS
    xt = feature.reshape(grid, _COLS, _CHUNK).transpose(0, 2, 1)
    out = pl.pallas_call(
        _expand_kernel,
        grid=(grid,),
        in_specs=[
            pl.BlockSpec((1, 1), lambda g: (0, 0), memory_space=pltpu.SMEM),
            pl.BlockSpec((1, 1), lambda g: (0, 0), memory_space=pltpu.SMEM),
            pl.BlockSpec((1, _CHUNK, _COLS), lambda g: (g, 0, 0)),
        ],
        out_specs=pl.BlockSpec((rows_per_step, _NUM_BINS + 1),
                               lambda g: (g, 0)),
        out_shape=jax.ShapeDtypeStruct((n, _NUM_BINS + 1), jnp.float32),
    )(lo, hi, xt)
    return out


# speedup vs baseline: 1.5680x; 1.0245x over previous
import jax
import jax.numpy as jnp
from jax.experimental import pallas as pl
from jax.experimental.pallas import tpu as pltpu

_NUM_BINS = 64
_CHUNK = 1024
_COLS = 32


def _minmax_kernel(x_ref, lo_ref, hi_ref):
    x = x_ref[...]
    lo_ref[0, 0] = jnp.min(x)
    hi_ref[0, 0] = jnp.max(x)


def _expand_kernel(lo_ref, hi_ref, xt_ref, out_ref):
    lo = lo_ref[0, 0]
    hi = hi_ref[0, 0]
    delta = (hi - lo) / jnp.float32(_NUM_BINS - 1)
    inv = jnp.float32(1.0) / delta
    xblk = xt_ref[0]
    t = (xblk - lo) * inv
    idx = jnp.clip(jnp.floor(t).astype(jnp.int32) + 1, 0, _NUM_BINS)
    cols = jax.lax.broadcasted_iota(jnp.int32, (_CHUNK, _NUM_BINS + 1), 1)
    for c in range(_COLS):
        icol = jax.lax.slice_in_dim(idx, c, c + 1, axis=1)
        out_ref[c * _CHUNK:(c + 1) * _CHUNK, :] = (
            icol == cols).astype(jnp.float32)


def kernel(feature):
    n = feature.shape[0]
    f2 = feature.reshape(n // 128, 128)
    lo, hi = pl.pallas_call(
        _minmax_kernel,
        out_shape=(
            jax.ShapeDtypeStruct((1, 1), jnp.float32),
            jax.ShapeDtypeStruct((1, 1), jnp.float32),
        ),
        out_specs=(
            pl.BlockSpec(memory_space=pltpu.SMEM),
            pl.BlockSpec(memory_space=pltpu.SMEM),
        ),
    )(f2)

    n_chunks = n // _CHUNK
    grid = n_chunks // _COLS
    rows_per_step = _CHUNK * _COLS
    xt = feature.reshape(grid, _COLS, _CHUNK).transpose(0, 2, 1)
    out = pl.pallas_call(
        _expand_kernel,
        grid=(grid,),
        in_specs=[
            pl.BlockSpec((1, 1), lambda g: (0, 0), memory_space=pltpu.SMEM),
            pl.BlockSpec((1, 1), lambda g: (0, 0), memory_space=pltpu.SMEM),
            pl.BlockSpec((1, _CHUNK, _COLS), lambda g: (g, 0, 0)),
        ],
        out_specs=pl.BlockSpec((rows_per_step, _NUM_BINS + 1),
                               lambda g: (g, 0)),
        out_shape=jax.ShapeDtypeStruct((n, _NUM_BINS + 1), jnp.float32),
    )(lo, hi, xt)
    return out
